# baseline (speedup 1.0000x reference)
import jax
import jax.numpy as jnp
from jax import lax
from jax.experimental import pallas as pl
from jax.experimental.pallas import tpu as pltpu
from jax.experimental.pallas import tpu_sc as plsc

_B = 4096
_BPW = 128
_W = 4
_MAIN = 896
_DP = 1024


def _idx_body(med_hbm, mole_hbm, idx_hbm, med_v, mole_v, idx_v):
    nc = 2
    wid = lax.axis_index("s") * nc + lax.axis_index("c")
    base = wid * _BPW
    pltpu.sync_copy(med_hbm, med_v)
    pltpu.sync_copy(mole_hbm.at[pl.ds(base, _BPW)], mole_v)
    for t in range(_BPW // 16):
        m16 = mole_v[pl.ds(t * 16, 16)]
        idx_v[pl.ds(t * 16, 16)] = plsc.load_gather(med_v, [m16])
    pltpu.sync_copy(idx_v, idx_hbm.at[pl.ds(base, _BPW)])


def _gather_body(table3_hbm, aux_hbm, idx_hbm, out_hbm,
                 idx_v, g_v, r_v, gsp, grp0, grp1, tail_v, stA, stB,
                 semA, semB, semt, semoA, semoB):
    nc = 2
    wid = lax.axis_index("s") * nc + lax.axis_index("c")
    base = wid * _BPW
    pltpu.sync_copy(idx_hbm.at[pl.ds(base, _BPW)], idx_v)
    for t in range(_BPW // 16):
        i16 = idx_v[pl.ds(t * 16, 16)]
        g16 = lax.shift_right_logical(i16, 3)
        g_v[pl.ds(t * 16, 16)] = g16
        r_v[pl.ds(t * 16, 16)] = lax.bitwise_and(i16, 7)
        p16 = jax.lax.broadcasted_iota(jnp.int32, (16,), 0) + t * 16
        d16 = lax.shift_right_logical(p16, 2) * 8 + lax.bitwise_and(p16, 3)
        plsc.store_scatter(gsp, [d16], g16)

    tail_cp = pltpu.make_async_copy(aux_hbm.at[idx_v], tail_v, semt)
    tail_cp.start()

    grps = (grp0, grp1)
    sems = (semA, semB)
    stg = (stA, stB)
    semos = (semoA, semoB)

    cp0 = pltpu.make_async_copy(
        table3_hbm.at[gsp.at[pl.ds(0, _W)], slice(None), pl.ds(0, _MAIN)],
        grps[0], sems[0])
    cp0.start()
    tail_cp.wait()

    n_waves = _BPW // _W
    waves_per_flush = 16 // _W

    def super_iter(si, _):
        r16a = r_v[pl.ds(si * 32, 16)]
        r16b = r_v[pl.ds(si * 32 + 16, 16)]
        r_scalars = ([r16a[j] for j in range(16)]
                     + [r16b[j] for j in range(16)])
        for wloc in range(8):
            w_dyn = si * 8 + wloc
            slot = wloc % 2
            pltpu.make_async_copy(
                table3_hbm.at[gsp.at[pl.ds(0, _W)], slice(None),
                              pl.ds(0, _MAIN)],
                grps[slot], sems[slot]).wait()
            @pl.when(w_dyn + 1 < n_waves)
            def _():
                pltpu.make_async_copy(
                    table3_hbm.at[gsp.at[pl.ds((w_dyn + 1) * 8, _W)],
                                  slice(None), pl.ds(0, _MAIN)],
                    grps[(wloc + 1) % 2], sems[(wloc + 1) % 2]).start()
            sslot = (wloc // waves_per_flush) % 2
            for k in range(_W):
                row_loc = wloc * _W + k
                srow = row_loc % 16
                r = r_scalars[row_loc]
                trow = si * 32 + row_loc

                def chunk_main(blk, _c, _k=k, _srow=srow, _r=r, _slot=slot):
                    c0 = blk * 16
                    stg[sslot][_srow, pl.ds(c0, 16)] = (
                        grps[_slot][_k, _r, pl.ds(c0, 16)])
                    return _c
                lax.fori_loop(0, _MAIN // 16, chunk_main, 0, unroll=8)
                for c0 in (896, 912, 928, 944, 960, 976, 984):
                    stg[sslot][srow, pl.ds(c0, 16)] = (
                        tail_v[trow, pl.ds(c0 - 872, 16)])
            if wloc % waves_per_flush == waves_per_flush - 1:
                @pl.when(si > 0)
                def _():
                    pltpu.make_async_copy(
                        stg[sslot], out_hbm.at[pl.ds(0, 16)],
                        semos[sslot]).wait()
                row0 = (wloc - waves_per_flush + 1) * _W
                pltpu.make_async_copy(
                    stg[sslot],
                    out_hbm.at[pl.ds(base + si * 32 + row0, 16)],
                    semos[sslot]).start()
        return 0

    lax.fori_loop(0, _BPW // 32, super_iter, 0)
    for sslot in range(2):
        pltpu.make_async_copy(stg[sslot], out_hbm.at[pl.ds(0, 16)],
                              semos[sslot]).wait()


def kernel(relation_matrix, med, mole):
    n, d = relation_matrix.shape
    b = mole.shape[0]
    table3 = relation_matrix.reshape(n // 8, 8, d)
    aux = lax.slice(relation_matrix, (0, d - 128), (n, d))
    mesh = plsc.VectorSubcoreMesh(core_axis_name="c", subcore_axis_name="s")
    idx = pl.kernel(
        _idx_body,
        mesh=mesh,
        compiler_params=pltpu.CompilerParams(needs_layout_passes=False),
        out_type=jax.ShapeDtypeStruct((b,), jnp.int32),
        scratch_types=[
            pltpu.VMEM((b,), jnp.int32),
            pltpu.VMEM((_BPW,), jnp.int32),
            pltpu.VMEM((_BPW,), jnp.int32),
        ],
    )(med, mole)
    out_padded = pl.kernel(
        _gather_body,
        mesh=mesh,
        compiler_params=pltpu.CompilerParams(needs_layout_passes=False),
        out_type=jax.ShapeDtypeStruct((b, _DP), relation_matrix.dtype),
        scratch_types=[
            pltpu.VMEM((_BPW,), jnp.int32),
            pltpu.VMEM((_BPW,), jnp.int32),
            pltpu.VMEM((_BPW,), jnp.int32),
            pltpu.VMEM((2 * _BPW,), jnp.int32),
            pltpu.VMEM((_W, 8, _MAIN), jnp.float32),
            pltpu.VMEM((_W, 8, _MAIN), jnp.float32),
            pltpu.VMEM((_BPW, 128), jnp.float32),
            pltpu.VMEM((16, _DP), jnp.float32),
            pltpu.VMEM((16, _DP), jnp.float32),
            pltpu.SemaphoreType.DMA,
            pltpu.SemaphoreType.DMA,
            pltpu.SemaphoreType.DMA,
            pltpu.SemaphoreType.DMA,
            pltpu.SemaphoreType.DMA,
        ],
    )(table3, aux, idx)
    return out_padded[:, :d]


# speedup vs baseline: 3.6671x; 3.6671x over previous
import jax
import jax.numpy as jnp
from jax import lax
from jax.experimental import pallas as pl
from jax.experimental.pallas import tpu as pltpu
from jax.experimental.pallas import tpu_sc as plsc

_BPW = 128
_X = 64
_CH = 32
_MAIN = 896
_TAIL = 104


def _sc_body(table_hbm, aux_hbm, med_hbm, mole_hbm, out_hbm, outt_hbm,
             med_v, mole_v, idx_v, rowA, rowB, mainbuf, tailbuf,
             semrA, semrB, semm, semt, semo1, semo2, semo3):
    nc = 2
    wid = lax.axis_index("s") * nc + lax.axis_index("c")
    base = wid * _BPW
    pltpu.sync_copy(med_hbm, med_v)
    pltpu.sync_copy(mole_hbm.at[pl.ds(base, _BPW)], mole_v)
    for t in range(_BPW // 16):
        m16 = mole_v[pl.ds(t * 16, 16)]
        idx_v[pl.ds(t * 16, 16)] = plsc.load_gather(med_v, [m16])

    tail_cp = pltpu.make_async_copy(
        aux_hbm.at[idx_v.at[pl.ds(_X, _BPW - _X)]], tailbuf, semt)
    tail_cp.start()
    main_cp = pltpu.make_async_copy(
        table_hbm.at[idx_v.at[pl.ds(_X, _CH)], pl.ds(0, _MAIN)],
        mainbuf, semm)
    main_cp.start()

    rowbufs = (rowA, rowB)
    semrs = (semrA, semrB)
    row_cps = [[], []]
    out_row_cps = [None, None]
    for c in range(_X // _CH):
        slot = c % 2
        for cp in row_cps[slot]:
            cp.wait()
        if out_row_cps[slot] is not None:
            out_row_cps[slot].wait()
        row_cps[slot] = []
        for t in range(_CH // 16):
            vblk = idx_v[pl.ds(c * _CH + t * 16, 16)]
            for jj in range(16):
                g = vblk[jj]
                cp = pltpu.make_async_copy(
                    table_hbm.at[g], rowbufs[slot].at[t * 16 + jj], semrs[slot])
                cp.start()
                row_cps[slot].append(cp)
    for slot in range(2):
        for cp in row_cps[slot]:
            cp.wait()
        oc = pltpu.make_async_copy(
            rowbufs[slot],
            out_hbm.at[pl.ds(base + slot * _CH, _CH)],
            semo1 if slot == 0 else semo2)
        oc.start()
        out_row_cps[slot] = oc

    main_cp.wait()
    oc_main1 = pltpu.make_async_copy(
        mainbuf, out_hbm.at[pl.ds(base + _X, _CH), pl.ds(0, _MAIN)], semo3)
    oc_main1.start()
    oc_main1.wait()
    main_cp2 = pltpu.make_async_copy(
        table_hbm.at[idx_v.at[pl.ds(_X + _CH, _CH)], pl.ds(0, _MAIN)],
        mainbuf, semm)
    main_cp2.start()
    main_cp2.wait()
    oc_main2 = pltpu.make_async_copy(
        mainbuf, out_hbm.at[pl.ds(base + _X + _CH, _CH), pl.ds(0, _MAIN)],
        semo3)
    oc_main2.start()

    tail_cp.wait()
    pltpu.sync_copy(tailbuf, outt_hbm.at[pl.ds(base + _X, _BPW - _X)])
    oc_main2.wait()
    for oc in out_row_cps:
        oc.wait()


def kernel(relation_matrix, med, mole):
    n, d = relation_matrix.shape
    b = mole.shape[0]
    aux = jnp.pad(lax.slice(relation_matrix, (0, _MAIN), (n, d)),
                  ((0, 0), (0, 128 - (d - _MAIN))))
    mesh = plsc.VectorSubcoreMesh(core_axis_name="c", subcore_axis_name="s")
    k = pl.kernel(
        _sc_body,
        mesh=mesh,
        compiler_params=pltpu.CompilerParams(needs_layout_passes=False),
        out_type=[jax.ShapeDtypeStruct((b, d), relation_matrix.dtype),
                  jax.ShapeDtypeStruct((b, 128), relation_matrix.dtype)],
        scratch_types=[
            pltpu.VMEM((b,), jnp.int32),
            pltpu.VMEM((_BPW,), jnp.int32),
            pltpu.VMEM((_BPW,), jnp.int32),
            pltpu.VMEM((_CH, 1000), jnp.float32),
            pltpu.VMEM((_CH, 1000), jnp.float32),
            pltpu.VMEM((_CH, _MAIN), jnp.float32),
            pltpu.VMEM((_BPW - _X, 128), jnp.float32),
            pltpu.SemaphoreType.DMA,
            pltpu.SemaphoreType.DMA,
            pltpu.SemaphoreType.DMA,
            pltpu.SemaphoreType.DMA,
            pltpu.SemaphoreType.DMA,
            pltpu.SemaphoreType.DMA,
            pltpu.SemaphoreType.DMA,
        ],
    )(relation_matrix, aux, med, mole)
    out_main, out_tail = k
    mask = ((jnp.arange(b) % _BPW) < _X)[:, None]
    tail_cols = jnp.where(mask, out_main[:, _MAIN:], out_tail[:, :_TAIL])
    return jnp.concatenate([out_main[:, :_MAIN], tail_cols], axis=1)


# speedup vs baseline: 4.5029x; 1.2279x over previous
import jax
import jax.numpy as jnp
from jax import lax
from jax.experimental import pallas as pl
from jax.experimental.pallas import tpu as pltpu
from jax.experimental.pallas import tpu_sc as plsc

_BPW = 128
_CHUNK = 32
_MAIN = 896
_TAIL = 104


def _sc_body(table_hbm, med_hbm, mole_hbm, out_hbm, med_v, mole_v, idx_v,
             bufA, bufB, tail_v, semA, semB, semt, semoA, semoB):
    nc = 2
    wid = lax.axis_index("s") * nc + lax.axis_index("c")
    base = wid * _BPW
    pltpu.sync_copy(med_hbm, med_v)
    pltpu.sync_copy(mole_hbm.at[pl.ds(base, _BPW)], mole_v)
    for t in range(_BPW // 16):
        m16 = mole_v[pl.ds(t * 16, 16)]
        idx_v[pl.ds(t * 16, 16)] = plsc.load_gather(med_v, [m16])

    tail_cps = []
    for t in range(_BPW // 16):
        vblk = idx_v[pl.ds(t * 16, 16)]
        for jj in range(16):
            g = vblk[jj]
            cp = pltpu.make_async_copy(
                table_hbm.at[g, pl.ds(_MAIN, _TAIL)],
                tail_v.at[t * 16 + jj], semt)
            cp.start()
            tail_cps.append(cp)

    bufs = (bufA, bufB)
    sems = (semA, semB)
    semos = (semoA, semoB)
    out_cps = [None, None]
    gathers = [None, None]
    n_chunks = _BPW // _CHUNK

    def fire(c):
        slot = c % 2
        cp = pltpu.make_async_copy(
            table_hbm.at[idx_v.at[pl.ds(c * _CHUNK, _CHUNK)],
                         pl.ds(0, _MAIN)],
            bufs[slot], sems[slot])
        cp.start()
        gathers[slot] = cp

    fire(0)
    for c in range(n_chunks):
        slot = c % 2
        gathers[slot].wait()
        if out_cps[slot] is not None:
            out_cps[slot].wait()
        oc = pltpu.make_async_copy(
            bufs[slot],
            out_hbm.at[pl.ds(base + c * _CHUNK, _CHUNK), pl.ds(0, _MAIN)],
            semos[slot])
        oc.start()
        out_cps[slot] = oc
        if c + 1 < n_chunks:
            nslot = (c + 1) % 2
            if out_cps[nslot] is not None:
                out_cps[nslot].wait()
                out_cps[nslot] = None
            fire(c + 1)
    for oc in out_cps:
        if oc is not None:
            oc.wait()

    for cp in tail_cps:
        cp.wait()
    pltpu.sync_copy(tail_v,
                    out_hbm.at[pl.ds(base, _BPW), pl.ds(_MAIN, _TAIL)])


def kernel(relation_matrix, med, mole):
    b = mole.shape[0]
    d = relation_matrix.shape[1]
    mesh = plsc.VectorSubcoreMesh(core_axis_name="c", subcore_axis_name="s")
    k = pl.kernel(
        _sc_body,
        mesh=mesh,
        compiler_params=pltpu.CompilerParams(needs_layout_passes=False),
        out_type=jax.ShapeDtypeStruct((b, d), relation_matrix.dtype),
        scratch_types=[
            pltpu.VMEM((b,), jnp.int32),
            pltpu.VMEM((_BPW,), jnp.int32),
            pltpu.VMEM((_BPW,), jnp.int32),
            pltpu.VMEM((_CHUNK, _MAIN), jnp.float32),
            pltpu.VMEM((_CHUNK, _MAIN), jnp.float32),
            pltpu.VMEM((_BPW, _TAIL), jnp.float32),
            pltpu.SemaphoreType.DMA,
            pltpu.SemaphoreType.DMA,
            pltpu.SemaphoreType.DMA,
            pltpu.SemaphoreType.DMA,
            pltpu.SemaphoreType.DMA,
        ],
    )
    return k(relation_matrix, med, mole)
